# baseline (speedup 1.0000x reference)
import jax
import jax.numpy as jnp
from jax.experimental import pallas as pl
from jax.experimental.pallas import tpu as pltpu


def _zeros(x):
    def body(x_ref, out_ref):
        out_ref[...] = jnp.zeros_like(out_ref)
    return pl.pallas_call(
        body,
        in_specs=[pl.BlockSpec(memory_space=pl.ANY)],
        out_specs=pl.BlockSpec(memory_space=pltpu.MemorySpace.VMEM),
        out_shape=jax.ShapeDtypeStruct((64, 128), jnp.float32),
    )(x)


def kernel(x, chans, coords):
    return _zeros(x)


# speedup vs baseline: 56.0147x; 56.0147x over previous
import functools

import jax
import jax.numpy as jnp
from jax import lax
from jax.experimental import pallas as pl
from jax.experimental.pallas import tpu as pltpu
from jax.experimental.pallas import tpu_sc as plsc


def _grab_units_sc(x2, chans, rows, cols, *, B, H, W):
    U = chans.shape[0]
    info = plsc.get_sparse_core_info()
    nw = info.num_cores * info.num_subcores
    bpt = B // nw
    mesh = plsc.VectorSubcoreMesh(core_axis_name="c", subcore_axis_name="s")

    @functools.partial(
        pl.kernel,
        mesh=mesh,
        out_type=jax.ShapeDtypeStruct((B * U,), jnp.float32),
        scratch_types=[
            pltpu.VMEM((U,), jnp.int32),
            pltpu.VMEM((U,), jnp.int32),
            pltpu.VMEM((U,), jnp.int32),
            pltpu.VMEM((bpt, U), jnp.int32),
            pltpu.VMEM((U, 128), jnp.float32),
            pltpu.VMEM((U,), jnp.float32),
            pltpu.SemaphoreType.DMA,
        ],
        compiler_params=pltpu.CompilerParams(
            use_tc_tiling_on_sc=True, needs_layout_passes=False
        ),
    )
    def k(x_hbm, ch_hbm, r_hbm, w_hbm, out_hbm, ch_v, r_v, c_v, idx_v,
          rowbuf, orow_v, sem):
        wid = lax.axis_index("s") * info.num_cores + lax.axis_index("c")
        pltpu.sync_copy(ch_hbm, ch_v)
        pltpu.sync_copy(r_hbm, r_v)
        pltpu.sync_copy(w_hbm, c_v)
        for lb in range(bpt):
            b = wid * bpt + lb
            for i in range(U // 16):
                s = pl.ds(16 * i, 16)
                idx_v[lb, s] = (b * H + r_v[s]) * W + c_v[s]
        for lb in range(bpt):
            b = wid * bpt + lb
            pltpu.async_copy(x_hbm.at[idx_v.at[lb]], rowbuf, sem).wait()
            for i in range(U // 16):
                s = pl.ds(16 * i, 16)
                rid = lax.iota(jnp.int32, 16) + 16 * i
                vals = plsc.load_gather(rowbuf, [rid, ch_v[s]])
                orow_v[s] = vals
            pltpu.sync_copy(orow_v, out_hbm.at[pl.ds(b * U, U)])

    out1d = k(x2, chans, rows, cols)
    return out1d.reshape(B, U)


def kernel(x, chans, coords):
    B, C, H, W = x.shape
    x2 = x.transpose(0, 2, 3, 1).reshape(B * H * W, C)
    ch = chans.astype(jnp.int32)
    r = coords[:, 0].astype(jnp.int32)
    c = coords[:, 1].astype(jnp.int32)
    return _grab_units_sc(x2, ch, r, c, B=B, H=H, W=W)


# speedup vs baseline: 57.2605x; 1.0222x over previous
import functools

import jax
import jax.numpy as jnp
from jax import lax
from jax.experimental import pallas as pl
from jax.experimental.pallas import tpu as pltpu
from jax.experimental.pallas import tpu_sc as plsc


def _grab_units_sc(x2, chans, rows, cols, *, B, H, W):
    U = chans.shape[0]
    info = plsc.get_sparse_core_info()
    nw = info.num_cores * info.num_subcores
    bpt = B // nw
    mesh = plsc.VectorSubcoreMesh(core_axis_name="c", subcore_axis_name="s")

    @functools.partial(
        pl.kernel,
        mesh=mesh,
        out_type=jax.ShapeDtypeStruct((B * U,), jnp.float32),
        scratch_types=[
            pltpu.VMEM((U,), jnp.int32),
            pltpu.VMEM((U,), jnp.int32),
            pltpu.VMEM((U,), jnp.int32),
            pltpu.VMEM((bpt, U), jnp.int32),
            pltpu.VMEM((bpt, U, 128), jnp.float32),
            pltpu.VMEM((U,), jnp.float32),
            pltpu.SemaphoreType.DMA((bpt,)),
        ],
        compiler_params=pltpu.CompilerParams(
            use_tc_tiling_on_sc=True, needs_layout_passes=False
        ),
    )
    def k(x_hbm, ch_hbm, r_hbm, w_hbm, out_hbm, ch_v, r_v, c_v, idx_v,
          rowbuf, orow_v, sem):
        wid = lax.axis_index("s") * info.num_cores + lax.axis_index("c")
        pltpu.sync_copy(ch_hbm, ch_v)
        pltpu.sync_copy(r_hbm, r_v)
        pltpu.sync_copy(w_hbm, c_v)
        for lb in range(bpt):
            b = wid * bpt + lb
            for i in range(U // 16):
                s = pl.ds(16 * i, 16)
                idx_v[lb, s] = (b * H + r_v[s]) * W + c_v[s]
        cps = [
            pltpu.async_copy(x_hbm.at[idx_v.at[lb]], rowbuf.at[lb], sem.at[lb])
            for lb in range(bpt)
        ]
        for lb in range(bpt):
            b = wid * bpt + lb
            cps[lb].wait()
            for i in range(U // 16):
                s = pl.ds(16 * i, 16)
                rid = lax.iota(jnp.int32, 16) + 16 * i
                vals = plsc.load_gather(rowbuf.at[lb], [rid, ch_v[s]])
                orow_v[s] = vals
            pltpu.sync_copy(orow_v, out_hbm.at[pl.ds(b * U, U)])

    out1d = k(x2, chans, rows, cols)
    return out1d.reshape(B, U)


def kernel(x, chans, coords):
    B, C, H, W = x.shape
    x2 = x.transpose(0, 2, 3, 1).reshape(B * H * W, C)
    ch = chans.astype(jnp.int32)
    r = coords[:, 0].astype(jnp.int32)
    c = coords[:, 1].astype(jnp.int32)
    return _grab_units_sc(x2, ch, r, c, B=B, H=H, W=W)
